# baseline (speedup 1.0000x reference)
import functools

import jax
import jax.numpy as jnp
from jax import lax
from jax.experimental import pallas as pl
from jax.experimental.pallas import tpu as pltpu
from jax.experimental.pallas import tpu_sc as plsc

B, L, D = 4096, 50, 128
N = B * L
NRANK, NSUIT = 13, 4
NCOMBO = NRANK * NSUIT
NCPAD = 56
NC, NS = 2, 16
NW = NC * NS
BATCH_PER_W = B // NW
ROWS_PER_W = N // NW
GCH = 64
NBUF = 4
NTILE = BATCH_PER_W // NBUF


def _combo_body(rank_ref, suit_ref, out_ref):
    rr = lax.broadcasted_iota(jnp.int32, (NCPAD, NRANK), 0) // NSUIT
    rc = lax.broadcasted_iota(jnp.int32, (NCPAD, NRANK), 1)
    oh_r = (rr == rc).astype(jnp.float32)
    sr = lax.broadcasted_iota(jnp.int32, (NCPAD, NSUIT), 0) % NSUIT
    sc = lax.broadcasted_iota(jnp.int32, (NCPAD, NSUIT), 1)
    oh_s = (sr == sc).astype(jnp.float32)
    out_ref[...] = (
        jnp.dot(oh_r, rank_ref[...], preferred_element_type=jnp.float32)
        + jnp.dot(oh_s, suit_ref[...], preferred_element_type=jnp.float32)
    )


def _make_combo(rank_emb, suit_emb):
    return pl.pallas_call(
        _combo_body,
        out_shape=jax.ShapeDtypeStruct((NCPAD, D), jnp.float32),
    )(rank_emb, suit_emb)


_SC_MESH = plsc.VectorSubcoreMesh(core_axis_name="c", subcore_axis_name="s")


@functools.partial(
    pl.kernel,
    mesh=_SC_MESH,
    compiler_params=pltpu.CompilerParams(
        needs_layout_passes=False, use_tc_tiling_on_sc=True),
    out_type=jax.ShapeDtypeStruct((B, 56, D), jnp.float32),
    scratch_types=[
        pltpu.VMEM((BATCH_PER_W, 2 * L), jnp.int32),
        pltpu.VMEM((BATCH_PER_W * GCH,), jnp.int32),
        *[pltpu.VMEM((GCH, D), jnp.float32) for _ in range(NBUF)],
        pltpu.VMEM_SHARED((NCPAD, D), jnp.float32),
        *[pltpu.SemaphoreType.DMA for _ in range(2 * NBUF)],
    ],
)
def _sc_lookup(cards_hbm, combo_hbm, out_hbm, cards_v, idx_v, *bufs):
    rows = bufs[:NBUF]
    combo_sp = bufs[NBUF]
    gsem = bufs[NBUF + 1:2 * NBUF + 1]
    ssem = bufs[2 * NBUF + 1:]
    sid = lax.axis_index("s")
    wid = sid * NC + lax.axis_index("c")
    bbase = wid * BATCH_PER_W
    rbase = pl.multiple_of(wid * ROWS_PER_W, ROWS_PER_W)

    @pl.when(sid == 0)
    def _():
        pltpu.sync_copy(combo_hbm, combo_sp)
    plsc.subcore_barrier()

    pltpu.sync_copy(cards_hbm.at[pl.ds(bbase, BATCH_PER_W)], cards_v)

    def idx_body(c, carry):
        goff = pl.multiple_of(c * GCH, GCH)
        b16 = jnp.full((16,), c, jnp.int32)
        for j in range(GCH // 16):
            l16 = lax.broadcasted_iota(jnp.int32, (16,), 0) + j * 16
            col = 2 * jnp.minimum(l16, L - 1)
            r16 = plsc.load_gather(cards_v, [b16, col])
            s16 = plsc.load_gather(cards_v, [b16, col + 1])
            idx_v[pl.ds(goff + j * 16, 16)] = r16 * NSUIT + s16
        return carry

    lax.fori_loop(0, BATCH_PER_W, idx_body, 0)

    def idx_ref(c):
        return idx_v.at[pl.ds(pl.multiple_of(c * GCH, GCH), GCH)]

    def start_gather(c, b):
        pltpu.make_async_copy(combo_sp.at[idx_ref(c)], rows[b], gsem[b]).start()

    def store(c, b):
        return pltpu.make_async_copy(rows[b].at[pl.ds(0, 56)],
                                     out_hbm.at[bbase + c], ssem[b])

    def ring_body(t, carry):
        base = t * NBUF
        for b in range(NBUF):
            @pl.when(t > 0)
            def _():
                store(base + b, b).wait()
            start_gather(base + b, b)
        for b in range(NBUF):
            pltpu.make_async_copy(combo_sp.at[idx_ref(base + b)],
                                  rows[b], gsem[b]).wait()
            store(base + b, b).start()
        return carry

    lax.fori_loop(0, NTILE, ring_body, 0)

    for b in range(NBUF):
        store(NTILE * NBUF - NBUF + b, b).wait()


def kernel(cards, rank_emb, suit_emb):
    combo = _make_combo(rank_emb, suit_emb)
    return _sc_lookup(cards.reshape(B, 2 * L), combo)[:, :L, :]


# speedup vs baseline: 1.2067x; 1.2067x over previous
import functools

import jax
import jax.numpy as jnp
from jax import lax
from jax.experimental import pallas as pl
from jax.experimental.pallas import tpu as pltpu
from jax.experimental.pallas import tpu_sc as plsc

B, L, D = 4096, 50, 128
N = B * L
NRANK, NSUIT = 13, 4
NCOMBO = NRANK * NSUIT
NCPAD = 56
NC, NS = 2, 16
NW = NC * NS
BATCH_PER_W = B // NW
ROWS_PER_W = N // NW
GCH = 56
NBUF = 4
NTILE = BATCH_PER_W // NBUF


def _combo_body(rank_ref, suit_ref, out_ref):
    rr = lax.broadcasted_iota(jnp.int32, (NCPAD, NRANK), 0) // NSUIT
    rc = lax.broadcasted_iota(jnp.int32, (NCPAD, NRANK), 1)
    oh_r = (rr == rc).astype(jnp.float32)
    sr = lax.broadcasted_iota(jnp.int32, (NCPAD, NSUIT), 0) % NSUIT
    sc = lax.broadcasted_iota(jnp.int32, (NCPAD, NSUIT), 1)
    oh_s = (sr == sc).astype(jnp.float32)
    out_ref[...] = (
        jnp.dot(oh_r, rank_ref[...], preferred_element_type=jnp.float32)
        + jnp.dot(oh_s, suit_ref[...], preferred_element_type=jnp.float32)
    )


def _make_combo(rank_emb, suit_emb):
    return pl.pallas_call(
        _combo_body,
        out_shape=jax.ShapeDtypeStruct((NCPAD, D), jnp.float32),
    )(rank_emb, suit_emb)


_SC_MESH = plsc.VectorSubcoreMesh(core_axis_name="c", subcore_axis_name="s")


@functools.partial(
    pl.kernel,
    mesh=_SC_MESH,
    compiler_params=pltpu.CompilerParams(
        needs_layout_passes=False, use_tc_tiling_on_sc=True),
    out_type=jax.ShapeDtypeStruct((B, L, D), jnp.float32),
    scratch_types=[
        pltpu.VMEM((BATCH_PER_W, 2 * L), jnp.int32),
        pltpu.VMEM((BATCH_PER_W * GCH + 16,), jnp.int32),
        *[pltpu.VMEM((GCH, D), jnp.float32) for _ in range(NBUF)],
        pltpu.VMEM_SHARED((NCPAD, D), jnp.float32),
        *[pltpu.SemaphoreType.DMA for _ in range(2 * NBUF)],
    ],
)
def _sc_lookup(cards_hbm, combo_hbm, out_hbm, cards_v, idx_v, *bufs):
    rows = bufs[:NBUF]
    combo_sp = bufs[NBUF]
    gsem = bufs[NBUF + 1:2 * NBUF + 1]
    ssem = bufs[2 * NBUF + 1:]
    sid = lax.axis_index("s")
    wid = sid * NC + lax.axis_index("c")
    bbase = wid * BATCH_PER_W
    rbase = pl.multiple_of(wid * ROWS_PER_W, ROWS_PER_W)

    @pl.when(sid == 0)
    def _():
        pltpu.sync_copy(combo_hbm, combo_sp)
    plsc.subcore_barrier()

    pltpu.sync_copy(cards_hbm.at[pl.ds(bbase, BATCH_PER_W)], cards_v)

    def idx_body(c, carry):
        goff = pl.multiple_of(c * GCH, GCH)
        b16 = jnp.full((16,), c, jnp.int32)
        for j in range(4):
            l16 = lax.broadcasted_iota(jnp.int32, (16,), 0) + j * 16
            col = 2 * jnp.minimum(l16, L - 1)
            r16 = plsc.load_gather(cards_v, [b16, col])
            s16 = plsc.load_gather(cards_v, [b16, col + 1])
            idx_v[pl.ds(goff + j * 16, 16)] = r16 * NSUIT + s16
        return carry

    lax.fori_loop(0, BATCH_PER_W, idx_body, 0)

    def idx_ref(c):
        return idx_v.at[pl.ds(pl.multiple_of(c * GCH, GCH), GCH)]

    def start_gather(c, b):
        pltpu.make_async_copy(combo_sp.at[idx_ref(c)], rows[b], gsem[b]).start()

    def store(c, b):
        return pltpu.make_async_copy(rows[b].at[pl.ds(0, L)],
                                     out_hbm.at[bbase + c], ssem[b])

    def ring_body(t, carry):
        base = t * NBUF
        for b in range(NBUF):
            @pl.when(t > 0)
            def _():
                store(base + b, b).wait()
            start_gather(base + b, b)
        for b in range(NBUF):
            pltpu.make_async_copy(combo_sp.at[idx_ref(base + b)],
                                  rows[b], gsem[b]).wait()
            store(base + b, b).start()
        return carry

    lax.fori_loop(0, NTILE, ring_body, 0)

    for b in range(NBUF):
        store(NTILE * NBUF - NBUF + b, b).wait()


def kernel(cards, rank_emb, suit_emb):
    combo = _make_combo(rank_emb, suit_emb)
    return _sc_lookup(cards.reshape(B, 2 * L), combo)


# speedup vs baseline: 1.2148x; 1.0067x over previous
import functools

import jax
import jax.numpy as jnp
from jax import lax
from jax.experimental import pallas as pl
from jax.experimental.pallas import tpu as pltpu
from jax.experimental.pallas import tpu_sc as plsc

B, L, D = 4096, 50, 128
N = B * L
NRANK, NSUIT = 13, 4
NCOMBO = NRANK * NSUIT
NCPAD = 56
NC, NS = 2, 16
NW = NC * NS
BATCH_PER_W = B // NW
ROWS_PER_W = N // NW
GCH = 56
NBUF = 8
NTILE = BATCH_PER_W // NBUF


def _combo_body(rank_ref, suit_ref, out_ref):
    rr = lax.broadcasted_iota(jnp.int32, (NCPAD, NRANK), 0) // NSUIT
    rc = lax.broadcasted_iota(jnp.int32, (NCPAD, NRANK), 1)
    oh_r = (rr == rc).astype(jnp.float32)
    sr = lax.broadcasted_iota(jnp.int32, (NCPAD, NSUIT), 0) % NSUIT
    sc = lax.broadcasted_iota(jnp.int32, (NCPAD, NSUIT), 1)
    oh_s = (sr == sc).astype(jnp.float32)
    out_ref[...] = (
        jnp.dot(oh_r, rank_ref[...], preferred_element_type=jnp.float32)
        + jnp.dot(oh_s, suit_ref[...], preferred_element_type=jnp.float32)
    )


def _make_combo(rank_emb, suit_emb):
    return pl.pallas_call(
        _combo_body,
        out_shape=jax.ShapeDtypeStruct((NCPAD, D), jnp.float32),
    )(rank_emb, suit_emb)


_SC_MESH = plsc.VectorSubcoreMesh(core_axis_name="c", subcore_axis_name="s")


@functools.partial(
    pl.kernel,
    mesh=_SC_MESH,
    compiler_params=pltpu.CompilerParams(
        needs_layout_passes=False, use_tc_tiling_on_sc=True),
    out_type=jax.ShapeDtypeStruct((B, L, D), jnp.float32),
    scratch_types=[
        pltpu.VMEM((BATCH_PER_W, 2 * L), jnp.int32),
        pltpu.VMEM((BATCH_PER_W * GCH + 16,), jnp.int32),
        *[pltpu.VMEM((GCH, D), jnp.float32) for _ in range(NBUF)],
        pltpu.VMEM_SHARED((NCPAD, D), jnp.float32),
        *[pltpu.SemaphoreType.DMA for _ in range(2 * NBUF)],
    ],
)
def _sc_lookup(cards_hbm, combo_hbm, out_hbm, cards_v, idx_v, *bufs):
    rows = bufs[:NBUF]
    combo_sp = bufs[NBUF]
    gsem = bufs[NBUF + 1:2 * NBUF + 1]
    ssem = bufs[2 * NBUF + 1:]
    sid = lax.axis_index("s")
    wid = sid * NC + lax.axis_index("c")
    bbase = wid * BATCH_PER_W
    rbase = pl.multiple_of(wid * ROWS_PER_W, ROWS_PER_W)

    @pl.when(sid == 0)
    def _():
        pltpu.sync_copy(combo_hbm, combo_sp)
    plsc.subcore_barrier()

    pltpu.sync_copy(cards_hbm.at[pl.ds(bbase, BATCH_PER_W)], cards_v)

    def idx_body(c, carry):
        goff = pl.multiple_of(c * GCH, GCH)
        b16 = jnp.full((16,), c, jnp.int32)
        for j in range(4):
            l16 = lax.broadcasted_iota(jnp.int32, (16,), 0) + j * 16
            col = 2 * jnp.minimum(l16, L - 1)
            r16 = plsc.load_gather(cards_v, [b16, col])
            s16 = plsc.load_gather(cards_v, [b16, col + 1])
            idx_v[pl.ds(goff + j * 16, 16)] = r16 * NSUIT + s16
        return carry

    lax.fori_loop(0, BATCH_PER_W, idx_body, 0)

    def idx_ref(c):
        return idx_v.at[pl.ds(pl.multiple_of(c * GCH, GCH), GCH)]

    def start_gather(c, b):
        pltpu.make_async_copy(combo_sp.at[idx_ref(c)], rows[b], gsem[b]).start()

    def store(c, b):
        return pltpu.make_async_copy(rows[b].at[pl.ds(0, L)],
                                     out_hbm.at[bbase + c], ssem[b])

    def ring_body(t, carry):
        base = t * NBUF
        for b in range(NBUF):
            @pl.when(t > 0)
            def _():
                store(base + b, b).wait()
            start_gather(base + b, b)
        for b in range(NBUF):
            pltpu.make_async_copy(combo_sp.at[idx_ref(base + b)],
                                  rows[b], gsem[b]).wait()
            store(base + b, b).start()
        return carry

    lax.fori_loop(0, NTILE, ring_body, 0)

    for b in range(NBUF):
        store(NTILE * NBUF - NBUF + b, b).wait()


def kernel(cards, rank_emb, suit_emb):
    combo = _make_combo(rank_emb, suit_emb)
    return _sc_lookup(cards.reshape(B, 2 * L), combo)
